# baseline (speedup 1.0000x reference)
import functools

import jax
import jax.numpy as jnp
from jax import lax
from jax.experimental import pallas as pl
from jax.experimental.pallas import tpu as pltpu
from jax.experimental.pallas import tpu_sc as plsc

D = 1000
V = 1000
B_TOTAL = 20480
NW = 32
B_PER_W = B_TOTAL // NW
CHUNK = 32
NCHUNK = B_PER_W // CHUNK


def _sc_gather(table, idx):
    mesh = plsc.VectorSubcoreMesh(core_axis_name="c", subcore_axis_name="s")

    @functools.partial(
        pl.kernel,
        mesh=mesh,
        compiler_params=pltpu.CompilerParams(use_tc_tiling_on_sc=False),
        out_type=jax.ShapeDtypeStruct((B_TOTAL, D), jnp.float32),
        scratch_types=[
            pltpu.VMEM((B_PER_W,), jnp.int32),
            pltpu.VMEM((2, CHUNK, D), jnp.float32),
            pltpu.VMEM_SHARED((V, D), jnp.float32),
            pltpu.SemaphoreType.DMA,
            pltpu.SemaphoreType.DMA,
            pltpu.SemaphoreType.DMA,
            pltpu.SemaphoreType.DMA,
        ],
    )
    def k(table_hbm, idx_hbm, out_hbm, idx_v, rows_v, table_s, g0, g1, s0, s1):
        sid = lax.axis_index("s")
        wid = sid * 2 + lax.axis_index("c")
        base = wid * B_PER_W
        gsem = (g0, g1)
        ssem = (s0, s1)

        @pl.when(sid < 8)
        def _stage():
            vbase = sid * 125
            pltpu.sync_copy(
                table_hbm.at[pl.ds(vbase, 125)], table_s.at[pl.ds(vbase, 125)]
            )

        pltpu.sync_copy(idx_hbm.at[pl.ds(base, B_PER_W)], idx_v)
        plsc.subcore_barrier()

        def gather(c, b):
            return pltpu.async_copy(
                table_s.at[idx_v.at[pl.ds(c * CHUNK, CHUNK)]],
                rows_v.at[b],
                gsem[b],
            )

        gathers = [gather(0, 0), None]
        scatters = [None, None]
        for c in range(NCHUNK):
            b = c % 2
            gathers[b].wait()
            if c + 1 < NCHUNK:
                nb = (c + 1) % 2
                if scatters[nb] is not None:
                    scatters[nb].wait()
                gathers[nb] = gather(c + 1, nb)
            scatters[b] = pltpu.async_copy(
                rows_v.at[b],
                out_hbm.at[pl.ds(base + c * CHUNK, CHUNK)],
                ssem[b],
            )
        scatters[0].wait()
        scatters[1].wait()

    return k(table, idx)


def kernel(x, y, table):
    idx = x.reshape(-1).astype(jnp.int32)
    out = _sc_gather(table, idx)
    one = jnp.where(y[0, 0] < jnp.int32(2**30), jnp.float32(1.0), jnp.float32(0.0))
    return out.reshape(x.shape[0], x.shape[1], D) * one


# speedup vs baseline: 1.5371x; 1.5371x over previous
import functools

import jax
import jax.numpy as jnp
from jax import lax
from jax.experimental import pallas as pl
from jax.experimental.pallas import tpu as pltpu
from jax.experimental.pallas import tpu_sc as plsc

D = 1000
V = 1000
B_TOTAL = 20480
NW = 32
B_PER_W = B_TOTAL // NW
CHUNK = 32
NCHUNK = B_PER_W // CHUNK


def _sc_gather(table, idx):
    mesh = plsc.VectorSubcoreMesh(core_axis_name="c", subcore_axis_name="s")

    @functools.partial(
        pl.kernel,
        mesh=mesh,
        compiler_params=pltpu.CompilerParams(use_tc_tiling_on_sc=False),
        out_type=jax.ShapeDtypeStruct((B_TOTAL, D), jnp.float32),
        scratch_types=[
            pltpu.VMEM((B_PER_W,), jnp.int32),
            pltpu.VMEM((2, CHUNK, D), jnp.float32),
            pltpu.VMEM_SHARED((V, D), jnp.float32),
            pltpu.SemaphoreType.DMA,
            pltpu.SemaphoreType.DMA,
            pltpu.SemaphoreType.DMA,
            pltpu.SemaphoreType.DMA,
        ],
    )
    def k(table_hbm, idx_hbm, out_hbm, idx_v, rows_v, table_s, g0, g1, s0, s1):
        sid = lax.axis_index("s")
        wid = sid * 2 + lax.axis_index("c")
        base = wid * B_PER_W
        gsem = (g0, g1)
        ssem = (s0, s1)

        @pl.when(sid < 8)
        def _stage():
            vbase = sid * 125
            pltpu.sync_copy(
                table_hbm.at[pl.ds(vbase, 125)], table_s.at[pl.ds(vbase, 125)]
            )

        pltpu.sync_copy(idx_hbm.at[pl.ds(base, B_PER_W)], idx_v)
        plsc.subcore_barrier()

        def gather(c, b):
            src = table_s if c % 2 == 0 else table_hbm
            return pltpu.async_copy(
                src.at[idx_v.at[pl.ds(c * CHUNK, CHUNK)]],
                rows_v.at[b],
                gsem[b],
            )

        gathers = [gather(0, 0), None]
        scatters = [None, None]
        for c in range(NCHUNK):
            b = c % 2
            gathers[b].wait()
            if c + 1 < NCHUNK:
                nb = (c + 1) % 2
                if scatters[nb] is not None:
                    scatters[nb].wait()
                gathers[nb] = gather(c + 1, nb)
            scatters[b] = pltpu.async_copy(
                rows_v.at[b],
                out_hbm.at[pl.ds(base + c * CHUNK, CHUNK)],
                ssem[b],
            )
        scatters[0].wait()
        scatters[1].wait()

    return k(table, idx)


def kernel(x, y, table):
    idx = x.reshape(-1).astype(jnp.int32)
    out = _sc_gather(table, idx)
    return out.reshape(x.shape[0], x.shape[1], D)


# speedup vs baseline: 1.6648x; 1.0831x over previous
import functools

import jax
import jax.numpy as jnp
from jax import lax
from jax.experimental import pallas as pl
from jax.experimental.pallas import tpu as pltpu
from jax.experimental.pallas import tpu_sc as plsc

D = 1000
V = 1000
B_TOTAL = 20480
NW = 32
B_PER_W = B_TOTAL // NW
CHUNK = 32
NCHUNK = B_PER_W // CHUNK


def _sc_gather(table, idx):
    mesh = plsc.VectorSubcoreMesh(core_axis_name="c", subcore_axis_name="s")

    @functools.partial(
        pl.kernel,
        mesh=mesh,
        compiler_params=pltpu.CompilerParams(use_tc_tiling_on_sc=False),
        out_type=jax.ShapeDtypeStruct((B_TOTAL, D), jnp.float32),
        scratch_types=[
            pltpu.VMEM((B_PER_W,), jnp.int32),
            pltpu.VMEM((2, CHUNK, D), jnp.float32),
            pltpu.VMEM_SHARED((V, D), jnp.float32),
            pltpu.SemaphoreType.DMA,
            pltpu.SemaphoreType.DMA,
            pltpu.SemaphoreType.DMA,
            pltpu.SemaphoreType.DMA,
        ],
    )
    def k(table_hbm, idx_hbm, out_hbm, idx_v, rows_v, table_s, g0, g1, s0, s1):
        sid = lax.axis_index("s")
        wid = sid * 2 + lax.axis_index("c")
        base = wid * B_PER_W
        gsem = (g0, g1)
        ssem = (s0, s1)

        @pl.when(sid < 8)
        def _stage():
            vbase = sid * 125
            pltpu.sync_copy(
                table_hbm.at[pl.ds(vbase, 125)], table_s.at[pl.ds(vbase, 125)]
            )

        pltpu.sync_copy(idx_hbm.at[pl.ds(base, B_PER_W)], idx_v)
        plsc.subcore_barrier()

        def gather(c, b):
            return pltpu.async_copy(
                table_s.at[idx_v.at[pl.ds(c * CHUNK, CHUNK)]],
                rows_v.at[b],
                gsem[b],
            )

        gathers = [gather(0, 0), None]
        scatters = [None, None]
        for c in range(NCHUNK):
            b = c % 2
            gathers[b].wait()
            if c + 1 < NCHUNK:
                nb = (c + 1) % 2
                if scatters[nb] is not None:
                    scatters[nb].wait()
                gathers[nb] = gather(c + 1, nb)
            scatters[b] = pltpu.async_copy(
                rows_v.at[b],
                out_hbm.at[pl.ds(base + c * CHUNK, CHUNK)],
                ssem[b],
            )
        scatters[0].wait()
        scatters[1].wait()

    return k(table, idx)


def kernel(x, y, table):
    idx = x.reshape(-1).astype(jnp.int32)
    out = _sc_gather(table, idx)
    return out.reshape(x.shape[0], x.shape[1], D)


# speedup vs baseline: 1.6678x; 1.0018x over previous
import functools

import jax
import jax.numpy as jnp
from jax import lax
from jax.experimental import pallas as pl
from jax.experimental.pallas import tpu as pltpu
from jax.experimental.pallas import tpu_sc as plsc

D = 1000
V = 1000
B_TOTAL = 20480
NW = 32
B_PER_W = B_TOTAL // NW
CHUNK = 32
NCHUNK = B_PER_W // CHUNK


def _sc_gather(table, idx):
    mesh = plsc.VectorSubcoreMesh(core_axis_name="c", subcore_axis_name="s")

    @functools.partial(
        pl.kernel,
        mesh=mesh,
        compiler_params=pltpu.CompilerParams(use_tc_tiling_on_sc=False),
        out_type=jax.ShapeDtypeStruct((B_TOTAL, D), jnp.float32),
        scratch_types=[
            pltpu.VMEM((B_PER_W,), jnp.int32),
            pltpu.VMEM((2, CHUNK, D), jnp.float32),
            pltpu.VMEM_SHARED((V, D), jnp.float32),
            pltpu.SemaphoreType.DMA,
            pltpu.SemaphoreType.DMA,
            pltpu.SemaphoreType.DMA,
            pltpu.SemaphoreType.DMA,
        ],
    )
    def k(table_hbm, idx_hbm, out_hbm, idx_v, rows_v, table_s, g0, g1, s0, s1):
        sid = lax.axis_index("s")
        wid = sid * 2 + lax.axis_index("c")
        base = wid * B_PER_W
        gsem = (g0, g1)
        ssem = (s0, s1)

        @pl.when(sid < 15)
        def _stage():
            vbase = sid * 63
            pltpu.sync_copy(
                table_hbm.at[pl.ds(vbase, 63)], table_s.at[pl.ds(vbase, 63)]
            )

        @pl.when(sid == 15)
        def _stage_tail():
            pltpu.sync_copy(
                table_hbm.at[pl.ds(945, 55)], table_s.at[pl.ds(945, 55)]
            )

        pltpu.sync_copy(idx_hbm.at[pl.ds(base, B_PER_W)], idx_v)
        plsc.subcore_barrier()

        def gather(c, b):
            return pltpu.async_copy(
                table_s.at[idx_v.at[pl.ds(c * CHUNK, CHUNK)]],
                rows_v.at[b],
                gsem[b],
            )

        gathers = [gather(0, 0), None]
        scatters = [None, None]
        for c in range(NCHUNK):
            b = c % 2
            gathers[b].wait()
            if c + 1 < NCHUNK:
                nb = (c + 1) % 2
                if scatters[nb] is not None:
                    scatters[nb].wait()
                gathers[nb] = gather(c + 1, nb)
            scatters[b] = pltpu.async_copy(
                rows_v.at[b],
                out_hbm.at[pl.ds(base + c * CHUNK, CHUNK)],
                ssem[b],
            )
        scatters[0].wait()
        scatters[1].wait()

    return k(table, idx)


def kernel(x, y, table):
    idx = x.reshape(-1).astype(jnp.int32)
    out = _sc_gather(table, idx)
    return out.reshape(x.shape[0], x.shape[1], D)
